# baseline (speedup 1.0000x reference)
import functools
import jax
import jax.numpy as jnp
from jax import lax
from jax.experimental import pallas as pl
from jax.experimental.pallas import tpu as pltpu
from jax.experimental.pallas import tpu_sc as plsc

_M = 64
_NPRIM = 2048
_NW = 32
_PB = 16
_NBATCH = _NPRIM // _NW // _PB


def _dist_body(xt_ref, d_ref):
    px = xt_ref[0]
    py = xt_ref[1]
    pz = xt_ref[2]
    BP = px.shape[0]

    def diffsq(a):
        d = a[:, :, None] - a[:, None, :]
        return d * d

    d2 = diffsq(px) + diffsq(py) + diffsq(pz)
    iu = lax.broadcasted_iota(jnp.int32, (BP, _M, _M), 1)
    ij = lax.broadcasted_iota(jnp.int32, (BP, _M, _M), 2)
    d2 = d2 + jnp.where(iu == ij, jnp.float32(1e9), jnp.float32(0.0))
    d_ref[...] = jnp.sqrt(jnp.maximum(d2, jnp.float32(1e-12)))


def _dist_tc(xt3, interpret=False):
    BP = 128
    grid = _NPRIM // BP
    return pl.pallas_call(
        _dist_body,
        grid=(grid,),
        in_specs=[pl.BlockSpec((3, BP, _M), lambda i: (0, i, 0))],
        out_specs=pl.BlockSpec((BP, _M, _M), lambda i: (i, 0, 0)),
        out_shape=jax.ShapeDtypeStruct((_NPRIM, _M, _M), jnp.float32),
        interpret=interpret,
    )(xt3)


def _prim_sc_body(d_hbm, par_hbm, el_hbm, dloc, bdm, bp, par_o, el_o):
    wid = lax.axis_index("s") * 2 + lax.axis_index("c")
    lanes = jnp.arange(16, dtype=jnp.int32)
    zeros = jnp.zeros((16,), jnp.int32)
    fnan = jnp.full((16,), jnp.nan, jnp.float32)
    finf = jnp.full((16,), jnp.inf, jnp.float32)
    neg1 = jnp.full((16,), -1, jnp.int32)
    fzero = jnp.zeros((16,), jnp.float32)
    pbase = lanes * (_M * _M)
    obase = lanes * _M

    for b in range(_NBATCH):
        base = wid * (_PB * _NBATCH) + b * _PB
        pltpu.sync_copy(d_hbm.at[pl.ds(base * _M * _M, _PB * _M * _M)], dloc)

        m0 = finf
        i0 = zeros
        for j in range(_M):
            jj = jnp.full((16,), j, jnp.int32)
            dj = plsc.load_gather(dloc, [pbase + j])
            if j == 0:
                dj = fnan
            bdm[pl.ds(j * 16, 16)] = dj
            bp[pl.ds(j * 16, 16)] = zeros
            par_o[pl.ds(j * 16, 16)] = neg1
            el_o[pl.ds(j * 16, 16)] = fzero
            c = dj < m0
            m0 = jnp.where(c, dj, m0)
            i0 = jnp.where(c, jj, i0)

        def step(t, carry):
            mval, u = carry
            su = u * 16 + lanes
            ou = obase + u
            bpu = plsc.load_gather(bp, [su])
            plsc.store_scatter(par_o, [ou], bpu)
            plsc.store_scatter(el_o, [ou], mval)
            plsc.store_scatter(bdm, [su], fnan)
            ubase = pbase + u * _M
            ma = finf
            ia = zeros
            mb = finf
            ib = zeros
            for j in range(0, _M, 2):
                j0 = jnp.full((16,), j, jnp.int32)
                j1 = jnp.full((16,), j + 1, jnp.int32)
                du0 = plsc.load_gather(dloc, [ubase + j])
                du1 = plsc.load_gather(dloc, [ubase + (j + 1)])
                bv0 = bdm[pl.ds(j * 16, 16)]
                bv1 = bdm[pl.ds((j + 1) * 16, 16)]
                upd0 = du0 < bv0
                upd1 = du1 < bv1
                nb0 = jnp.where(upd0, du0, bv0)
                nb1 = jnp.where(upd1, du1, bv1)
                bdm[pl.ds(j * 16, 16)] = nb0
                bdm[pl.ds((j + 1) * 16, 16)] = nb1
                bp[pl.ds(j * 16, 16)] = jnp.where(upd0, u, bp[pl.ds(j * 16, 16)])
                bp[pl.ds((j + 1) * 16, 16)] = jnp.where(upd1, u, bp[pl.ds((j + 1) * 16, 16)])
                ca = nb0 < ma
                ma = jnp.where(ca, nb0, ma)
                ia = jnp.where(ca, j0, ia)
                cb = nb1 < mb
                mb = jnp.where(cb, nb1, mb)
                ib = jnp.where(cb, j1, ib)
            cc = (mb < ma) | ((mb == ma) & (ib < ia))
            m = jnp.where(cc, mb, ma)
            idx = jnp.where(cc, ib, ia)
            return (m, idx)

        lax.fori_loop(1, _M, step, (m0, i0))

        pltpu.sync_copy(par_o, par_hbm.at[pl.ds(base * _M, _PB * _M)])
        pltpu.sync_copy(el_o, el_hbm.at[pl.ds(base * _M, _PB * _M)])


def _prim_sc(d):
    mesh = plsc.VectorSubcoreMesh(core_axis_name="c", subcore_axis_name="s")
    f = functools.partial(
        pl.kernel,
        out_type=(
            jax.ShapeDtypeStruct((_NPRIM * _M,), jnp.int32),
            jax.ShapeDtypeStruct((_NPRIM * _M,), jnp.float32),
        ),
        mesh=mesh,
        compiler_params=pltpu.CompilerParams(needs_layout_passes=False),
        scratch_types=[
            pltpu.VMEM((_PB * _M * _M,), jnp.float32),
            pltpu.VMEM((_M * 16,), jnp.float32),
            pltpu.VMEM((_M * 16,), jnp.int32),
            pltpu.VMEM((_PB * _M,), jnp.int32),
            pltpu.VMEM((_PB * _M,), jnp.float32),
        ],
    )(_prim_sc_body)
    par, el = f(d.reshape(_NPRIM * _M * _M))
    return par.reshape(_NPRIM, _M), el.reshape(_NPRIM, _M)


def _epi_body(par_ref, el_ref, alpha_ref, dist_ref, asg_ref, mean_ref):
    el = el_ref[...]
    par = par_ref[...]
    mean = jnp.sum(el, axis=1, keepdims=True) / jnp.float32(_M - 1)
    alpha = alpha_ref[0, 0]
    penal = el > alpha * mean
    dist_ref[...] = jnp.where(penal, el, jnp.float32(0.0))
    i0 = lax.broadcasted_iota(jnp.int32, (_NPRIM, _M), 0)
    offs = (i0 % 128) * _M
    asg_ref[...] = jnp.where(penal & (par >= 0), par + offs, jnp.int32(-1))
    mean_ref[...] = jnp.broadcast_to(mean, (_NPRIM, 8))


def _epi_tc(par, el, alpha_vec, interpret=False):
    return pl.pallas_call(
        _epi_body,
        out_shape=(
            jax.ShapeDtypeStruct((_NPRIM, _M), jnp.float32),
            jax.ShapeDtypeStruct((_NPRIM, _M), jnp.int32),
            jax.ShapeDtypeStruct((_NPRIM, 8), jnp.float32),
        ),
        interpret=interpret,
    )(par, el, alpha_vec)


def kernel(input, primitive_size, alpha):
    x = input.astype(jnp.float32)
    B, n, _ = x.shape
    G = n // _M
    xt3 = x.reshape(B * G, _M, 3).transpose(2, 0, 1)
    d = _dist_tc(xt3)
    par, el = _prim_sc(d)
    alpha_vec = jnp.full((8, 128), alpha, jnp.float32)
    dist, asg, mean8 = _epi_tc(par, el, alpha_vec)
    mean_mst_length = jnp.sum(mean8[:, 0].reshape(B, G), axis=1)
    return (dist.reshape(B, n), asg.reshape(B, n),
            mean_mst_length / (n / primitive_size))
